# baseline (speedup 1.0000x reference)
import functools

import jax
import jax.numpy as jnp
from jax import lax
from jax.experimental import pallas as pl
from jax.experimental.pallas import tpu as pltpu
from jax.experimental.pallas import tpu_sc as plsc

CLS = 1_000_000
FEAT = 32
BATCH = 16384
NC = 2
NS = 16
L = 16
NW = NC * NS
B_W = BATCH // NW
REGION = 62528
TBL = NS * REGION
ZCH = 15632
IDX_CH = 128
W_SCALE = 2.0 * FEAT * CLS


def _body(xs_h, ys_h, cen_h, out_h,
          table, zbuf, ones_v, ysc, ysb, xsv, rows, cntv, partv, sem):
    c = lax.axis_index("c")
    s = lax.axis_index("s")
    wid = s * NC + c

    zero16 = jnp.zeros((L,), jnp.float32)
    one16 = jnp.ones((L,), jnp.float32)

    def zfill(i, carry):
        zbuf[pl.ds(i * L, L)] = zero16
        return carry

    lax.fori_loop(0, ZCH // L, zfill, 0)
    for k in range(IDX_CH // L):
        ones_v[pl.ds(k * L, L)] = one16

    for k in range(REGION // ZCH):
        off = pl.multiple_of(s * REGION + k * ZCH, 8)
        pltpu.sync_copy(zbuf, table.at[pl.ds(off, ZCH)])

    pltpu.sync_copy(ys_h.at[pl.ds(s * 8, 8)], ysc)
    plsc.subcore_barrier()

    for j in range(ysc.shape[0]):
        pltpu.sync_copy(ones_v, table.at[ysc.at[j]], add=True)
    plsc.subcore_barrier()

    pltpu.sync_copy(ys_h.at[pl.ds(wid * 4, 4)], ysb)
    pltpu.sync_copy(xs_h.at[wid], xsv)
    copies = [
        pltpu.async_copy(cen_h.at[ysb.at[j]],
                         rows.at[pl.ds(j * IDX_CH, IDX_CH)], sem)
        for j in range(ysb.shape[0])
    ]
    for j in range(ysb.shape[0]):
        pltpu.sync_copy(table.at[ysb.at[j]], cntv.at[pl.ds(j * IDX_CH, IDX_CH)])
    for cp in copies:
        cp.wait()

    def grp(g, total):
        r0 = g * L
        c16 = cntv[pl.ds(r0, L)]
        w16 = 1.0 / (c16 * W_SCALE)
        acc = total
        for k in range(L):
            r = r0 + k
            wk = jnp.full((L,), w16[k])
            x0 = xsv[pl.ds(r * FEAT, L)]
            x1 = xsv[pl.ds(r * FEAT + L, L)]
            c0 = rows[r, pl.ds(0, L)]
            c1 = rows[r, pl.ds(L, L)]
            d0 = x0 - c0
            d1 = x1 - c1
            acc = acc + (d0 * d0 + d1 * d1) * wk
        return acc

    part = lax.fori_loop(0, B_W // L, grp, jnp.zeros((L,), jnp.float32))
    partv[...] = part
    pltpu.sync_copy(partv, out_h.at[wid])


@jax.jit
def kernel(xs, ys, center):
    ys2 = ys.reshape(BATCH // IDX_CH, IDX_CH)
    xs_flat = xs.reshape(NW, B_W * FEAT)
    run = functools.partial(
        pl.kernel,
        mesh=plsc.VectorSubcoreMesh(core_axis_name="c", subcore_axis_name="s"),
        out_type=jax.ShapeDtypeStruct((NW, L), jnp.float32),
        scratch_types=[
            pltpu.VMEM_SHARED((TBL,), jnp.float32),
            pltpu.VMEM((ZCH,), jnp.float32),
            pltpu.VMEM((IDX_CH,), jnp.float32),
            pltpu.VMEM((BATCH // NS // IDX_CH, IDX_CH), jnp.int32),
            pltpu.VMEM((B_W // IDX_CH, IDX_CH), jnp.int32),
            pltpu.VMEM((B_W * FEAT,), jnp.float32),
            pltpu.VMEM((B_W, FEAT), jnp.float32),
            pltpu.VMEM((B_W,), jnp.float32),
            pltpu.VMEM((L,), jnp.float32),
            pltpu.SemaphoreType.DMA,
        ],
        compiler_params=pltpu.CompilerParams(use_tc_tiling_on_sc=False),
    )(_body)
    out = run(xs_flat, ys2, center)
    return jnp.sum(out)
